# baseline (speedup 1.0000x reference)
import functools
import math

import jax
import jax.numpy as jnp
from jax.experimental import pallas as pl
from jax.experimental.pallas import tpu as pltpu

_BASE = 10000.0
_NEG_LN_BASE = -math.log(_BASE)
_GRP = 128


def _rope_cache_kernel(cos_ref, sin_ref, base_c_ref, base_s_ref, *, rows, dim):
    i = pl.program_id(0)
    half = dim // 2
    groups = rows // _GRP

    j = jax.lax.broadcasted_iota(jnp.int32, (_GRP, dim), 1)
    jm = jnp.where(j < half, j, j - half).astype(jnp.float32)
    inv_freq = jnp.exp(jm * (_NEG_LN_BASE / half))

    r = jax.lax.broadcasted_iota(jnp.int32, (_GRP, dim), 0).astype(jnp.float32)

    @pl.when(i == 0)
    def _build_base():
        ang_r = r * inv_freq
        base_c_ref[...] = jnp.cos(ang_r)
        base_s_ref[...] = jnp.sin(ang_r)

    base_c = base_c_ref[...]
    base_s = base_s_ref[...]

    step_base = (i * rows).astype(jnp.float32)
    ang_g = (r * float(_GRP) + step_base) * inv_freq
    grp_c = jnp.cos(ang_g)
    grp_s = jnp.sin(ang_g)

    for k in range(groups):
        gc = jax.lax.slice(grp_c, (k, 0), (k + 1, dim))
        gs = jax.lax.slice(grp_s, (k, 0), (k + 1, dim))
        lo = k * _GRP
        cos_ref[lo:lo + _GRP, :] = base_c * gc - base_s * gs
        sin_ref[lo:lo + _GRP, :] = base_s * gc + base_c * gs


def kernel(x):
    seq, dim = x.shape[2], x.shape[3]
    rows = seq // 2
    grid = (seq // rows,)
    body = functools.partial(_rope_cache_kernel, rows=rows, dim=dim)
    cos, sin = pl.pallas_call(
        body,
        grid=grid,
        out_specs=[pl.BlockSpec((rows, dim), lambda i: (i, 0))] * 2,
        out_shape=[jax.ShapeDtypeStruct((seq, dim), x.dtype)] * 2,
        scratch_shapes=[pltpu.VMEM((_GRP, dim), jnp.float32)] * 2,
    )()
    return (cos.astype(x.dtype), sin.astype(x.dtype))


# speedup vs baseline: 1.0444x; 1.0444x over previous
import functools
import math

import jax
import jax.numpy as jnp
from jax.experimental import pallas as pl
from jax.experimental.pallas import tpu as pltpu

_BASE = 10000.0
_NEG_LN_BASE = -math.log(_BASE)
_GRP = 64


def _rope_cache_kernel(cos_ref, sin_ref, base_c_ref, base_s_ref, *, rows, dim):
    i = pl.program_id(0)
    half = dim // 2
    groups = rows // _GRP

    j = jax.lax.broadcasted_iota(jnp.int32, (_GRP, dim), 1)
    jm = jnp.where(j < half, j, j - half).astype(jnp.float32)
    inv_freq = jnp.exp(jm * (_NEG_LN_BASE / half))

    r = jax.lax.broadcasted_iota(jnp.int32, (_GRP, dim), 0).astype(jnp.float32)

    @pl.when(i == 0)
    def _build_base():
        ang_r = r * inv_freq
        base_c_ref[...] = jnp.cos(ang_r)
        base_s_ref[...] = jnp.sin(ang_r)

    base_c = base_c_ref[...]
    base_s = base_s_ref[...]

    step_base = (i * rows).astype(jnp.float32)
    ang_g = (r * float(_GRP) + step_base) * inv_freq
    grp_c = jnp.cos(ang_g)
    grp_s = jnp.sin(ang_g)

    for k in range(groups):
        gc = jax.lax.slice(grp_c, (k, 0), (k + 1, dim))
        gs = jax.lax.slice(grp_s, (k, 0), (k + 1, dim))
        lo = k * _GRP
        cos_ref[lo:lo + _GRP, :] = base_c * gc - base_s * gs
        sin_ref[lo:lo + _GRP, :] = base_s * gc + base_c * gs


def kernel(x):
    seq, dim = x.shape[2], x.shape[3]
    rows = seq // 2
    grid = (seq // rows,)
    body = functools.partial(_rope_cache_kernel, rows=rows, dim=dim)
    cos, sin = pl.pallas_call(
        body,
        grid=grid,
        out_specs=[pl.BlockSpec((rows, dim), lambda i: (i, 0))] * 2,
        out_shape=[jax.ShapeDtypeStruct((seq, dim), x.dtype)] * 2,
        scratch_shapes=[pltpu.VMEM((_GRP, dim), jnp.float32)] * 2,
    )()
    return (cos.astype(x.dtype), sin.astype(x.dtype))
